# Initial kernel scaffold; baseline (speedup 1.0000x reference)
import functools

import jax
import jax.numpy as jnp
from jax import lax
from jax.experimental import pallas as pl
from jax.experimental.pallas import tpu as pltpu
from jax.experimental.pallas import tpu_sc as plsc

_INFO = plsc.get_sparse_core_info()
_NC = _INFO.num_cores
_NS = _INFO.num_subcores
_NW = _NC * _NS

_ROWS = 128


def _make_sc_gather(n_chunks_total: int, dim: int):
  chunks_pw = n_chunks_total // _NW
  nbuf = 4
  assert chunks_pw % nbuf == 0

  mesh = plsc.VectorSubcoreMesh(core_axis_name="c", subcore_axis_name="s")

  @functools.partial(
      pl.kernel,
      out_type=jax.ShapeDtypeStruct((n_chunks_total * _ROWS, dim),
                                    jnp.float32),
      mesh=mesh,
      scratch_types=[
          pltpu.VMEM((chunks_pw, _ROWS), jnp.int32),
          pltpu.VMEM((nbuf, _ROWS, dim), jnp.float32),
          pltpu.SemaphoreType.DMA,
      ],
  )
  def sc_gather(ids_hbm, table_hbm, out_hbm, idx_v, rows_v, gsem):
    wid = lax.axis_index("s") * _NC + lax.axis_index("c")
    chunk0 = wid * chunks_pw
    pltpu.sync_copy(ids_hbm.at[pl.ds(chunk0, chunks_pw)], idx_v)

    def fire(j, b):
      pltpu.async_copy(table_hbm.at[idx_v.at[j]], rows_v.at[b], gsem)

    for b in range(nbuf):
      fire(b, b)

    @pl.loop(0, chunks_pw, step=nbuf)
    def _(g0):
      for b in range(nbuf):
        j = g0 + b
        pltpu.make_async_copy(table_hbm.at[idx_v.at[j]], rows_v.at[b],
                              gsem).wait()
        pltpu.sync_copy(rows_v.at[b],
                        out_hbm.at[pl.ds((chunk0 + j) * _ROWS, _ROWS)])

        @pl.when(j + nbuf < chunks_pw)
        def _():
          fire(j + nbuf, b)

  return sc_gather


def kernel(input_ids, embedding_table):
  b, s = input_ids.shape
  _, dim = embedding_table.shape
  n = b * s
  ids = input_ids.reshape(n // _ROWS, _ROWS).astype(jnp.int32)
  gather = _make_sc_gather(n // _ROWS, dim)
  out = gather(ids, embedding_table)
  return out.reshape(b, s, dim)


# speedup vs baseline: 1.4938x; 1.4938x over previous
import functools

import jax
import jax.numpy as jnp
from jax import lax
from jax.experimental import pallas as pl
from jax.experimental.pallas import tpu as pltpu
from jax.experimental.pallas import tpu_sc as plsc

_INFO = plsc.get_sparse_core_info()
_NC = _INFO.num_cores
_NS = _INFO.num_subcores
_NW = _NC * _NS

_ROWS = 128


def _make_sc_gather(n_chunks_total: int, dim: int):
  chunks_pw = n_chunks_total // _NW
  nbuf = 4
  assert chunks_pw % nbuf == 0

  mesh = plsc.VectorSubcoreMesh(core_axis_name="c", subcore_axis_name="s")

  @functools.partial(
      pl.kernel,
      out_type=jax.ShapeDtypeStruct((n_chunks_total * _ROWS, dim),
                                    jnp.float32),
      mesh=mesh,
      compiler_params=pltpu.CompilerParams(use_tc_tiling_on_sc=False),
      scratch_types=[
          pltpu.VMEM((chunks_pw, _ROWS), jnp.int32),
          pltpu.VMEM((nbuf, _ROWS, dim), jnp.float32),
          pltpu.SemaphoreType.DMA,
      ],
  )
  def sc_gather(ids_hbm, table_hbm, out_hbm, idx_v, rows_v, gsem):
    wid = lax.axis_index("s") * _NC + lax.axis_index("c")
    chunk0 = wid * chunks_pw
    pltpu.sync_copy(ids_hbm.at[pl.ds(chunk0, chunks_pw)], idx_v)

    def fire(j, b):
      pltpu.async_copy(table_hbm.at[idx_v.at[j]], rows_v.at[b], gsem)

    for b in range(nbuf):
      fire(b, b)

    @pl.loop(0, chunks_pw, step=nbuf)
    def _(g0):
      for b in range(nbuf):
        j = g0 + b
        pltpu.make_async_copy(table_hbm.at[idx_v.at[j]], rows_v.at[b],
                              gsem).wait()
        pltpu.sync_copy(rows_v.at[b],
                        out_hbm.at[pl.ds((chunk0 + j) * _ROWS, _ROWS)])

        @pl.when(j + nbuf < chunks_pw)
        def _():
          fire(j + nbuf, b)

  return sc_gather


def kernel(input_ids, embedding_table):
  b, s = input_ids.shape
  _, dim = embedding_table.shape
  n = b * s
  ids = input_ids.reshape(n // _ROWS, _ROWS).astype(jnp.int32)
  gather = _make_sc_gather(n // _ROWS, dim)
  out = gather(ids, embedding_table)
  return out.reshape(b, s, dim)


# speedup vs baseline: 1.4986x; 1.0032x over previous
import functools

import jax
import jax.numpy as jnp
from jax import lax
from jax.experimental import pallas as pl
from jax.experimental.pallas import tpu as pltpu
from jax.experimental.pallas import tpu_sc as plsc

_INFO = plsc.get_sparse_core_info()
_NC = _INFO.num_cores
_NS = _INFO.num_subcores
_NW = _NC * _NS

_ROWS = 128


def _make_sc_gather(n_chunks_total: int, dim: int):
  chunks_pw = n_chunks_total // _NW
  nbuf = 10
  depth = 5
  assert chunks_pw % nbuf == 0

  mesh = plsc.VectorSubcoreMesh(core_axis_name="c", subcore_axis_name="s")

  @functools.partial(
      pl.kernel,
      out_type=jax.ShapeDtypeStruct((n_chunks_total * _ROWS, dim),
                                    jnp.float32),
      mesh=mesh,
      compiler_params=pltpu.CompilerParams(use_tc_tiling_on_sc=False),
      scratch_types=[
          pltpu.VMEM((chunks_pw, _ROWS), jnp.int32),
          pltpu.VMEM((nbuf, _ROWS, dim), jnp.float32),
          pltpu.SemaphoreType.DMA,
          pltpu.SemaphoreType.DMA,
      ],
  )
  def sc_gather(ids_hbm, table_hbm, out_hbm, idx_v, rows_v, gsem, osem):
    wid = lax.axis_index("s") * _NC + lax.axis_index("c")
    chunk0 = wid * chunks_pw
    pltpu.sync_copy(ids_hbm.at[pl.ds(chunk0, chunks_pw)], idx_v)

    def fire(j, b):
      pltpu.async_copy(table_hbm.at[idx_v.at[j]], rows_v.at[b], gsem)

    def drain_one_store():
      pltpu.make_async_copy(rows_v.at[0], out_hbm.at[pl.ds(0, _ROWS)],
                            osem).wait()

    for m in range(depth):
      fire(m, m)

    @pl.loop(0, chunks_pw, step=nbuf)
    def _(g0):
      for k in range(nbuf):
        j = g0 + k
        pltpu.make_async_copy(table_hbm.at[idx_v.at[j]], rows_v.at[k],
                              gsem).wait()
        pltpu.async_copy(rows_v.at[k],
                         out_hbm.at[pl.ds((chunk0 + j) * _ROWS, _ROWS)],
                         osem)
        m = j + depth
        bm = (k + depth) % nbuf

        @pl.when(jnp.logical_and(m >= nbuf, m < chunks_pw))
        def _():
          drain_one_store()

        @pl.when(m < chunks_pw)
        def _():
          fire(m, bm)

    for _ in range(nbuf):
      drain_one_store()

  return sc_gather


def kernel(input_ids, embedding_table):
  b, s = input_ids.shape
  _, dim = embedding_table.shape
  n = b * s
  ids = input_ids.reshape(n // _ROWS, _ROWS).astype(jnp.int32)
  gather = _make_sc_gather(n // _ROWS, dim)
  out = gather(ids, embedding_table)
  return out.reshape(b, s, dim)
